# baseline (speedup 1.0000x reference)
import dataclasses
import functools

import jax
import jax.numpy as jnp
from jax import lax
from jax.experimental import pallas as pl
from jax.experimental.pallas import tpu as pltpu
from jax.experimental.pallas import tpu_sc as plsc

N = 10000
F = 128
E = 320000

NC, NS, L = 2, 16, 16
NPAD = 10240
RPT = NPAD // NS
BPT = 80
NBLK = NC * NS * BPT
EPAD = NBLK * 128
CH = 8
BPT0 = 80
BPT1 = 80
NCH0 = BPT0 // CH
NCH1 = BPT1 // CH
NBLK0 = NS * BPT0

@functools.cache
def _sc_mesh():
    return plsc.VectorSubcoreMesh(
        core_axis_name="c", subcore_axis_name="s", num_cores=NC, num_subcores=NS
    )


@functools.cache
def _sc_params():
    cp = pltpu.CompilerParams()
    if "needs_layout_passes" in pltpu.CompilerParams.__dataclass_fields__:
        cp = dataclasses.replace(cp, needs_layout_passes=False)
    return cp


def _hist_sc(dst_blocks):
    @functools.partial(
        pl.kernel,
        out_type=jax.ShapeDtypeStruct((NC, NPAD), jnp.float32),
        mesh=_sc_mesh(),
        compiler_params=_sc_params(),
        scratch_types=[
            pltpu.VMEM((BPT, 128), jnp.int32),
            pltpu.VMEM((NPAD,), jnp.float32),
            pltpu.VMEM((NS, RPT), jnp.float32),
            pltpu.VMEM_SHARED((NS, NPAD), jnp.float32),
        ],
    )
    def k(dst_hbm, out_hbm, idx_v, hist_v, col_v, stage_sh):
        c = lax.axis_index("c")
        s = lax.axis_index("s")
        base_blk = (c * NS + s) * BPT
        pltpu.sync_copy(dst_hbm.at[pl.ds(base_blk, BPT)], idx_v)

        zeros16 = jnp.zeros((L,), jnp.float32)
        ones16 = jnp.ones((L,), jnp.float32)

        @pl.loop(0, NPAD // L)
        def _(i):
            hist_v[pl.ds(i * L, L)] = zeros16

        @pl.loop(0, BPT)
        def _(b):
            for j in range(128 // L):
                idx16 = idx_v[b, pl.ds(j * L, L)]
                plsc.addupdate_scatter(hist_v, [idx16], ones16)

        pltpu.sync_copy(hist_v, stage_sh.at[s])
        plsc.subcore_barrier()

        for r in range(NS):
            pltpu.sync_copy(stage_sh.at[r, pl.ds(s * RPT, RPT)], col_v.at[r])

        @pl.loop(0, RPT // L)
        def _(j):
            acc = col_v[0, pl.ds(j * L, L)]
            for r in range(1, NS):
                acc = acc + col_v[r, pl.ds(j * L, L)]
            col_v[0, pl.ds(j * L, L)] = acc

        pltpu.sync_copy(col_v.at[0], out_hbm.at[c, pl.ds(s * RPT, RPT)])

    return k(dst_blocks)


def _matmul_scale_tc(x_pad, W, histc):
    BM = 1024

    def body(x_ref, w_ref, h_ref, g_ref, d_ref):
        dinv = lax.rsqrt(1.0 + h_ref[...])
        h = lax.dot_general(
            x_ref[...], w_ref[...], (((1,), (1,)), ((), ())),
            precision=lax.Precision.HIGHEST,
            preferred_element_type=jnp.float32,
        )
        g_ref[...] = h * dinv
        d_ref[...] = dinv

    return pl.pallas_call(
        body,
        grid=(NPAD // BM,),
        in_specs=[
            pl.BlockSpec((BM, F), lambda i: (i, 0)),
            pl.BlockSpec((F, F), lambda i: (0, 0)),
            pl.BlockSpec((BM, 1), lambda i: (i, 0)),
        ],
        out_specs=[
            pl.BlockSpec((BM, F), lambda i: (i, 0)),
            pl.BlockSpec((BM, 1), lambda i: (i, 0)),
        ],
        out_shape=[
            jax.ShapeDtypeStruct((NPAD, F), jnp.float32),
            jax.ShapeDtypeStruct((NPAD, 1), jnp.float32),
        ],
    )(x_pad, W, histc)


def _scatter_sc(g_pad, src_blocks, dst_blocks):
    @functools.partial(
        pl.kernel,
        out_type=jax.ShapeDtypeStruct((NC, NPAD, F), jnp.float32),
        mesh=_sc_mesh(),
        compiler_params=_sc_params(),
        scratch_types=[
            [pltpu.VMEM((CH, 128), jnp.int32)] * 2,
            [pltpu.VMEM((CH, 128), jnp.int32)] * 2,
            [pltpu.VMEM((128, F), jnp.float32)] * 2,
            [pltpu.SemaphoreType.DMA] * 2,
            [pltpu.SemaphoreType.DMA] * 2,
            [pltpu.SemaphoreType.DMA] * 2,
            pltpu.VMEM_SHARED((NPAD, F), jnp.float32),
        ],
    )
    def k(g_hbm, src_hbm, dst_hbm, out_hbm, src_sl, dst_sl, bufs, gsems, ssems, isems, acc_sh):
        c = lax.axis_index("c")
        s = lax.axis_index("s")
        on0 = c == 0
        base_blk = jnp.where(on0, s * BPT0, NBLK0 + s * BPT1)
        nch = jnp.where(on0, NCH0, NCH1)
        pltpu.sync_copy(g_hbm.at[pl.ds(s * RPT, RPT)], acc_sh.at[pl.ds(s * RPT, RPT)])
        plsc.subcore_barrier()

        def gather_issue(slot, j, p):
            for h in (0, 64):
                pltpu.async_copy(
                    g_hbm.at[src_sl[slot].at[j, pl.ds(h, 64)]],
                    bufs[p].at[pl.ds(h, 64)], gsems[p],
                )

        def gather_wait(slot, j, p):
            for h in (0, 64):
                pltpu.make_async_copy(
                    g_hbm.at[src_sl[slot].at[j, pl.ds(h, 64)]],
                    bufs[p].at[pl.ds(h, 64)], gsems[p],
                ).wait()

        @pl.when(nch > 0)
        def _():
            pltpu.sync_copy(src_hbm.at[pl.ds(base_blk, CH)], src_sl[0])
            pltpu.sync_copy(dst_hbm.at[pl.ds(base_blk, CH)], dst_sl[0])
            gather_issue(0, 0, 0)

        @pl.loop(0, nch, step=2)
        def _(cc):
            for ss in (0, 1):
                ci = cc + ss
                for j in range(CH):
                    p = j % 2
                    q = 1 - p
                    gather_wait(ss, j, p)
                    pltpu.async_copy(
                        bufs[p], acc_sh.at[dst_sl[ss].at[j]], ssems[p], add=True
                    )
                    if j == 0:
                        @pl.when(ci > 0)
                        def _():
                            pltpu.make_async_copy(
                                bufs[q], acc_sh.at[dst_sl[1 - ss].at[CH - 1]], ssems[q]
                            ).wait()

                        @pl.when(ci + 1 < nch)
                        def _():
                            nb = base_blk + (ci + 1) * CH
                            pltpu.async_copy(
                                src_hbm.at[pl.ds(nb, CH)], src_sl[1 - ss], isems[1 - ss]
                            )
                            pltpu.async_copy(
                                dst_hbm.at[pl.ds(nb, CH)], dst_sl[1 - ss], isems[1 - ss]
                            )
                    else:
                        pltpu.make_async_copy(
                            bufs[q], acc_sh.at[dst_sl[ss].at[j - 1]], ssems[q]
                        ).wait()
                    if j < CH - 1:
                        gather_issue(ss, j + 1, q)
                    else:
                        @pl.when(ci + 1 < nch)
                        def _():
                            nb = base_blk + (ci + 1) * CH
                            pltpu.make_async_copy(
                                src_hbm.at[pl.ds(nb, CH)], src_sl[1 - ss], isems[1 - ss]
                            ).wait()
                            pltpu.make_async_copy(
                                dst_hbm.at[pl.ds(nb, CH)], dst_sl[1 - ss], isems[1 - ss]
                            ).wait()
                            gather_issue(1 - ss, 0, q)

        @pl.when(nch > 0)
        def _():
            pltpu.make_async_copy(
                bufs[1], acc_sh.at[dst_sl[1].at[CH - 1]], ssems[1]
            ).wait()

        plsc.subcore_barrier()
        pltpu.sync_copy(acc_sh.at[pl.ds(s * RPT, RPT)], out_hbm.at[c, pl.ds(s * RPT, RPT)])

    return k(g_pad, src_blocks, dst_blocks)


def _combine_stats_tc(g_pad, S2, dinv, b2):
    BM = 2000
    G = N // BM

    def body(g_ref, s_ref, d_ref, b_ref, pre_ref, sum_ref, sq_ref):
        i = pl.program_id(0)
        pre = (s_ref[0] + s_ref[1] - g_ref[...]) * d_ref[...] + b_ref[...]
        pre_ref[...] = pre
        p8 = jnp.sum(pre.reshape(BM // 8, 8, F), axis=0)
        q8 = jnp.sum((pre * pre).reshape(BM // 8, 8, F), axis=0)

        @pl.when(i == 0)
        def _():
            sum_ref[...] = p8
            sq_ref[...] = q8

        @pl.when(i > 0)
        def _():
            sum_ref[...] += p8
            sq_ref[...] += q8

    return pl.pallas_call(
        body,
        grid=(G,),
        in_specs=[
            pl.BlockSpec((BM, F), lambda i: (i, 0)),
            pl.BlockSpec((NC, BM, F), lambda i: (0, i, 0)),
            pl.BlockSpec((BM, 1), lambda i: (i, 0)),
            pl.BlockSpec((1, F), lambda i: (0, 0)),
        ],
        out_specs=[
            pl.BlockSpec((BM, F), lambda i: (i, 0)),
            pl.BlockSpec((8, F), lambda i: (0, 0)),
            pl.BlockSpec((8, F), lambda i: (0, 0)),
        ],
        out_shape=[
            jax.ShapeDtypeStruct((N, F), jnp.float32),
            jax.ShapeDtypeStruct((8, F), jnp.float32),
            jax.ShapeDtypeStruct((8, F), jnp.float32),
        ],
    )(g_pad, S2, dinv, b2)


def _bn_relu_tc(pre, s8, q8, gamma2, beta2):
    BM = 2000
    G = N // BM

    def body(pre_ref, s_ref, q_ref, g_ref, b_ref, out_ref):
        mean = jnp.sum(s_ref[...], axis=0, keepdims=True) / N
        ex2 = jnp.sum(q_ref[...], axis=0, keepdims=True) / N
        var = ex2 - mean * mean
        scale = g_ref[...] * lax.rsqrt(var + 1e-5)
        out_ref[...] = jnp.maximum((pre_ref[...] - mean) * scale + b_ref[...], 0.0)

    return pl.pallas_call(
        body,
        grid=(G,),
        in_specs=[
            pl.BlockSpec((BM, F), lambda i: (i, 0)),
            pl.BlockSpec((8, F), lambda i: (0, 0)),
            pl.BlockSpec((8, F), lambda i: (0, 0)),
            pl.BlockSpec((1, F), lambda i: (0, 0)),
            pl.BlockSpec((1, F), lambda i: (0, 0)),
        ],
        out_specs=pl.BlockSpec((BM, F), lambda i: (i, 0)),
        out_shape=jax.ShapeDtypeStruct((N, F), jnp.float32),
    )(pre, s8, q8, gamma2, beta2)


def kernel(x, edge_index, W, b, gamma, beta):
    src = edge_index[0].astype(jnp.int32)
    dst = edge_index[1].astype(jnp.int32)
    padi = N + (jnp.arange(EPAD - E, dtype=jnp.int32) % (NPAD - N))
    srcb = jnp.concatenate([src, padi]).reshape(NBLK, 128)
    dstb = jnp.concatenate([dst, padi]).reshape(NBLK, 128)

    hist2 = _hist_sc(dstb)
    histc = (hist2[0] + hist2[1]).reshape(NPAD, 1)

    x_pad = jnp.concatenate([x, jnp.zeros((NPAD - N, F), x.dtype)])
    g_pad, dinv = _matmul_scale_tc(x_pad, W, histc)

    S2 = _scatter_sc(g_pad, srcb, dstb)

    pre, s8, q8 = _combine_stats_tc(g_pad, S2, dinv, b.reshape(1, F))
    return _bn_relu_tc(pre, s8, q8, gamma.reshape(1, F), beta.reshape(1, F))


# speedup vs baseline: 1.0287x; 1.0287x over previous
import dataclasses
import functools

import jax
import jax.numpy as jnp
from jax import lax
from jax.experimental import pallas as pl
from jax.experimental.pallas import tpu as pltpu
from jax.experimental.pallas import tpu_sc as plsc

N = 10000
F = 128
E = 320000

NC, NS, L = 2, 16, 16
NPAD = 10240
RPT = NPAD // NS
BPT = 80
NBLK = NC * NS * BPT
EPAD = NBLK * 128
CH = 8
BPT0 = 80
BPT1 = 80
NCH0 = BPT0 // CH
NCH1 = BPT1 // CH
NBLK0 = NS * BPT0

@functools.cache
def _sc_mesh():
    return plsc.VectorSubcoreMesh(
        core_axis_name="c", subcore_axis_name="s", num_cores=NC, num_subcores=NS
    )


@functools.cache
def _sc_params():
    cp = pltpu.CompilerParams()
    if "needs_layout_passes" in pltpu.CompilerParams.__dataclass_fields__:
        cp = dataclasses.replace(cp, needs_layout_passes=False)
    return cp


def _hist_sc(dst_blocks):
    @functools.partial(
        pl.kernel,
        out_type=jax.ShapeDtypeStruct((NC, NPAD), jnp.float32),
        mesh=_sc_mesh(),
        compiler_params=_sc_params(),
        scratch_types=[
            pltpu.VMEM((BPT, 128), jnp.int32),
            pltpu.VMEM((NPAD,), jnp.float32),
            pltpu.VMEM((NS, RPT), jnp.float32),
            pltpu.VMEM_SHARED((NS, NPAD), jnp.float32),
        ],
    )
    def k(dst_hbm, out_hbm, idx_v, hist_v, col_v, stage_sh):
        c = lax.axis_index("c")
        s = lax.axis_index("s")
        base_blk = (c * NS + s) * BPT
        pltpu.sync_copy(dst_hbm.at[pl.ds(base_blk, BPT)], idx_v)

        zeros16 = jnp.zeros((L,), jnp.float32)
        ones16 = jnp.ones((L,), jnp.float32)

        @pl.loop(0, NPAD // L)
        def _(i):
            hist_v[pl.ds(i * L, L)] = zeros16

        @pl.loop(0, BPT)
        def _(b):
            for j in range(128 // L):
                idx16 = idx_v[b, pl.ds(j * L, L)]
                plsc.addupdate_scatter(hist_v, [idx16], ones16)

        pltpu.sync_copy(hist_v, stage_sh.at[s])
        plsc.subcore_barrier()

        for r in range(NS):
            pltpu.sync_copy(stage_sh.at[r, pl.ds(s * RPT, RPT)], col_v.at[r])

        @pl.loop(0, RPT // L)
        def _(j):
            acc = col_v[0, pl.ds(j * L, L)]
            for r in range(1, NS):
                acc = acc + col_v[r, pl.ds(j * L, L)]
            col_v[0, pl.ds(j * L, L)] = acc

        pltpu.sync_copy(col_v.at[0], out_hbm.at[c, pl.ds(s * RPT, RPT)])

    return k(dst_blocks)


def _matmul_scale_tc(x_pad, W, histc):
    BM = 1024

    def body(x_ref, w_ref, h_ref, g_ref, d_ref):
        dinv = lax.rsqrt(1.0 + h_ref[...])
        h = lax.dot_general(
            x_ref[...], w_ref[...], (((1,), (1,)), ((), ())),
            precision=lax.Precision.HIGHEST,
            preferred_element_type=jnp.float32,
        )
        g_ref[...] = h * dinv
        d_ref[...] = dinv

    return pl.pallas_call(
        body,
        grid=(NPAD // BM,),
        in_specs=[
            pl.BlockSpec((BM, F), lambda i: (i, 0)),
            pl.BlockSpec((F, F), lambda i: (0, 0)),
            pl.BlockSpec((BM, 1), lambda i: (i, 0)),
        ],
        out_specs=[
            pl.BlockSpec((BM, F), lambda i: (i, 0)),
            pl.BlockSpec((BM, 1), lambda i: (i, 0)),
        ],
        out_shape=[
            jax.ShapeDtypeStruct((NPAD, F), jnp.float32),
            jax.ShapeDtypeStruct((NPAD, 1), jnp.float32),
        ],
    )(x_pad, W, histc)


def _scatter_sc(g_pad, src_blocks, dst_blocks):
    @functools.partial(
        pl.kernel,
        out_type=jax.ShapeDtypeStruct((NC, NPAD, F), jnp.float32),
        mesh=_sc_mesh(),
        compiler_params=_sc_params(),
        scratch_types=[
            [pltpu.VMEM((CH, 128), jnp.int32)] * 2,
            [pltpu.VMEM((CH, 128), jnp.int32)] * 2,
            [pltpu.VMEM((128, F), jnp.float32)] * 2,
            [pltpu.SemaphoreType.DMA] * 2,
            [pltpu.SemaphoreType.DMA] * 2,
            [pltpu.SemaphoreType.DMA] * 2,
            pltpu.VMEM_SHARED((NPAD, F), jnp.float32),
        ],
    )
    def k(g_hbm, src_hbm, dst_hbm, out_hbm, src_sl, dst_sl, bufs, gsems, ssems, isems, acc_sh):
        c = lax.axis_index("c")
        s = lax.axis_index("s")
        on0 = c == 0
        base_blk = jnp.where(on0, s * BPT0, NBLK0 + s * BPT1)
        nch = jnp.where(on0, NCH0, NCH1)
        pltpu.sync_copy(g_hbm.at[pl.ds(s * RPT, RPT)], acc_sh.at[pl.ds(s * RPT, RPT)])
        plsc.subcore_barrier()

        def gather_issue(slot, j, p):
            for h in (0, 64):
                pltpu.async_copy(
                    g_hbm.at[src_sl[slot].at[j, pl.ds(h, 64)]],
                    bufs[p].at[pl.ds(h, 64)], gsems[p],
                )

        def gather_wait(slot, j, p):
            for h in (0, 64):
                pltpu.make_async_copy(
                    g_hbm.at[src_sl[slot].at[j, pl.ds(h, 64)]],
                    bufs[p].at[pl.ds(h, 64)], gsems[p],
                ).wait()

        @pl.when(nch > 0)
        def _():
            pltpu.sync_copy(src_hbm.at[pl.ds(base_blk, CH)], src_sl[0])
            pltpu.sync_copy(dst_hbm.at[pl.ds(base_blk, CH)], dst_sl[0])
            gather_issue(0, 0, 0)

        @pl.loop(0, nch, step=2)
        def _(cc):
            for ss in (0, 1):
                ci = cc + ss
                for j in range(CH):
                    p = j % 2
                    q = 1 - p
                    gather_wait(ss, j, p)
                    pltpu.async_copy(
                        bufs[p], acc_sh.at[dst_sl[ss].at[j]], ssems[p], add=True
                    )
                    if j == 0:
                        @pl.when(ci > 0)
                        def _():
                            pltpu.make_async_copy(
                                bufs[q], acc_sh.at[dst_sl[1 - ss].at[CH - 1]], ssems[q]
                            ).wait()

                        @pl.when(ci + 1 < nch)
                        def _():
                            nb = base_blk + (ci + 1) * CH
                            pltpu.async_copy(
                                src_hbm.at[pl.ds(nb, CH)], src_sl[1 - ss], isems[1 - ss]
                            )
                            pltpu.async_copy(
                                dst_hbm.at[pl.ds(nb, CH)], dst_sl[1 - ss], isems[1 - ss]
                            )
                    else:
                        pltpu.make_async_copy(
                            bufs[q], acc_sh.at[dst_sl[ss].at[j - 1]], ssems[q]
                        ).wait()
                    if j < CH - 1:
                        gather_issue(ss, j + 1, q)
                    else:
                        @pl.when(ci + 1 < nch)
                        def _():
                            nb = base_blk + (ci + 1) * CH
                            pltpu.make_async_copy(
                                src_hbm.at[pl.ds(nb, CH)], src_sl[1 - ss], isems[1 - ss]
                            ).wait()
                            pltpu.make_async_copy(
                                dst_hbm.at[pl.ds(nb, CH)], dst_sl[1 - ss], isems[1 - ss]
                            ).wait()
                            gather_issue(1 - ss, 0, q)

        @pl.when(nch > 0)
        def _():
            pltpu.make_async_copy(
                bufs[1], acc_sh.at[dst_sl[1].at[CH - 1]], ssems[1]
            ).wait()

        plsc.subcore_barrier()
        pltpu.sync_copy(acc_sh.at[pl.ds(s * RPT, RPT)], out_hbm.at[c, pl.ds(s * RPT, RPT)])

    return k(g_pad, src_blocks, dst_blocks)


def _combine_bn_tc(g_pad, S2, dinv, b2, gamma2, beta2):
    BM = 2000
    G = N // BM

    def body(g_ref, s_ref, d_ref, b_ref, gm_ref, bt_ref, out_ref,
             pre_v, sum_v, sq_v):
        i = pl.program_id(0)
        pre = (s_ref[0] + s_ref[1] - g_ref[...]) * d_ref[...] + b_ref[...]
        pre_v[pl.ds(i * BM, BM), :] = pre
        p8 = jnp.sum(pre.reshape(BM // 8, 8, F), axis=0)
        q8 = jnp.sum((pre * pre).reshape(BM // 8, 8, F), axis=0)

        @pl.when(i == 0)
        def _():
            sum_v[...] = p8
            sq_v[...] = q8

        @pl.when(i > 0)
        def _():
            sum_v[...] += p8
            sq_v[...] += q8

        @pl.when(i == G - 1)
        def _():
            mean = jnp.sum(sum_v[...], axis=0, keepdims=True) / N
            ex2 = jnp.sum(sq_v[...], axis=0, keepdims=True) / N
            var = ex2 - mean * mean
            scale = gm_ref[...] * lax.rsqrt(var + 1e-5)
            out_ref[...] = jnp.maximum(
                (pre_v[...] - mean) * scale + bt_ref[...], 0.0
            )

    return pl.pallas_call(
        body,
        grid=(G,),
        in_specs=[
            pl.BlockSpec((BM, F), lambda i: (i, 0)),
            pl.BlockSpec((NC, BM, F), lambda i: (0, i, 0)),
            pl.BlockSpec((BM, 1), lambda i: (i, 0)),
            pl.BlockSpec((1, F), lambda i: (0, 0)),
            pl.BlockSpec((1, F), lambda i: (0, 0)),
            pl.BlockSpec((1, F), lambda i: (0, 0)),
        ],
        out_specs=pl.BlockSpec((N, F), lambda i: (0, 0)),
        out_shape=jax.ShapeDtypeStruct((N, F), jnp.float32),
        scratch_shapes=[
            pltpu.VMEM((N, F), jnp.float32),
            pltpu.VMEM((8, F), jnp.float32),
            pltpu.VMEM((8, F), jnp.float32),
        ],
    )(g_pad, S2, dinv, b2, gamma2, beta2)


def kernel(x, edge_index, W, b, gamma, beta):
    src = edge_index[0].astype(jnp.int32)
    dst = edge_index[1].astype(jnp.int32)
    padi = N + (jnp.arange(EPAD - E, dtype=jnp.int32) % (NPAD - N))
    srcb = jnp.concatenate([src, padi]).reshape(NBLK, 128)
    dstb = jnp.concatenate([dst, padi]).reshape(NBLK, 128)

    hist2 = _hist_sc(dstb)
    histc = (hist2[0] + hist2[1]).reshape(NPAD, 1)

    x_pad = jnp.concatenate([x, jnp.zeros((NPAD - N, F), x.dtype)])
    g_pad, dinv = _matmul_scale_tc(x_pad, W, histc)

    S2 = _scatter_sc(g_pad, srcb, dstb)

    return _combine_bn_tc(
        g_pad, S2, dinv,
        b.reshape(1, F), gamma.reshape(1, F), beta.reshape(1, F),
    )
